# baseline (speedup 1.0000x reference)
import functools

import jax
import jax.numpy as jnp
from jax import lax
from jax.experimental import pallas as pl
from jax.experimental.pallas import tpu as pltpu
from jax.experimental.pallas import tpu_sc as plsc

B, C, D = 16, 512, 4096
LANES = 16
NW = 32
ROWS = B * C
CH = 4
RPW = ROWS // NW
NCH = RPW // CH


def _gather_add(x2d, gidx3):
    mesh = plsc.VectorSubcoreMesh(core_axis_name="c", subcore_axis_name="s")

    @functools.partial(
        pl.kernel,
        out_type=jax.ShapeDtypeStruct((ROWS, D), jnp.float32),
        mesh=mesh,
        scratch_types=[
            pltpu.VMEM((2 * NCH, CH // 2), jnp.int32),
            [[pltpu.VMEM((CH // 2, D), jnp.float32) for _ in range(2)]
             for _ in range(2)],
            [[pltpu.VMEM((CH // 2, D), jnp.float32) for _ in range(2)]
             for _ in range(2)],
            [pltpu.VMEM((CH, D), jnp.float32) for _ in range(2)],
            [[pltpu.SemaphoreType.DMA for _ in range(2)] for _ in range(2)],
            [[pltpu.SemaphoreType.DMA for _ in range(2)] for _ in range(2)],
            [pltpu.SemaphoreType.DMA for _ in range(2)],
        ],
    )
    def sc_kernel(x_hbm, gidx_hbm, out_hbm, idx_v, gbufs, ibufs, obufs,
                  sem_g, sem_i, sem_o):
        wid = lax.axis_index("s") * 2 + lax.axis_index("c")
        wbase = wid * RPW
        pltpu.sync_copy(gidx_hbm.at[wid], idx_v)
        HC = CH // 2

        def start_in(k, slot):
            for h in range(2):
                pltpu.async_copy(x_hbm.at[idx_v.at[2 * k + h]],
                                 gbufs[slot][h], sem_g[slot][h])
                pltpu.async_copy(
                    x_hbm.at[pl.ds(wbase + k * CH + h * HC, HC)],
                    ibufs[slot][h], sem_i[slot][h]
                )

        def wait_in(slot):
            for h in range(2):
                pltpu.make_async_copy(x_hbm.at[pl.ds(0, HC)],
                                      gbufs[slot][h], sem_g[slot][h]).wait()
                pltpu.make_async_copy(x_hbm.at[pl.ds(0, HC)],
                                      ibufs[slot][h], sem_i[slot][h]).wait()

        def add_chunk(slot):
            obuf = obufs[slot]

            def add_body(j, c2):
                off = j * LANES
                for r in range(CH):
                    h, hr = r // HC, r % HC
                    obuf[r, pl.ds(off, LANES)] = (
                        ibufs[slot][h][hr, pl.ds(off, LANES)]
                        + gbufs[slot][h][hr, pl.ds(off, LANES)]
                    )
                return c2

            lax.fori_loop(0, D // LANES, add_body, 0)

        def start_out(k, slot):
            pltpu.async_copy(
                obufs[slot], out_hbm.at[pl.ds(wbase + k * CH, CH)], sem_o[slot]
            )

        def wait_out(k, slot):
            pltpu.make_async_copy(obufs[slot],
                                  out_hbm.at[pl.ds(wbase + k * CH, CH)],
                                  sem_o[slot]).wait()

        start_in(0, 0)
        start_in(1, 1)

        def pipe_body(i, carry):
            k0 = i * 2

            def step(k, slot):
                wait_in(slot)

                @pl.when(k >= 2)
                def _():
                    wait_out(k - 2, slot)

                add_chunk(slot)
                start_out(k, slot)

                @pl.when(k + 2 < NCH)
                def _():
                    start_in(k + 2, slot)

            step(k0, 0)
            step(k0 + 1, 1)
            return carry

        lax.fori_loop(0, NCH // 2, pipe_body, 0)
        wait_out(NCH - 2, 0)
        wait_out(NCH - 1, 1)

    return sc_kernel(x2d, gidx3)


def kernel(x, W1, b1, W2, b2):
    pooled = jnp.mean(x, axis=2)
    h = pooled @ W1 + b1
    h = jnp.where(h > 0, h, 0.01 * h)
    scores = jax.nn.sigmoid(h @ W2 + b2)
    ch_order = jnp.argsort(-scores, axis=1)

    gidx = ch_order.astype(jnp.int32) + (jnp.arange(B, dtype=jnp.int32) * C)[:, None]
    out2d = _gather_add(x.reshape(ROWS, D), gidx.reshape(NW, 2 * NCH, CH // 2))
    return out2d.reshape(B, C, D)


# speedup vs baseline: 2.0053x; 2.0053x over previous
import functools

import jax
import jax.numpy as jnp
from jax import lax
from jax.experimental import pallas as pl
from jax.experimental.pallas import tpu as pltpu
from jax.experimental.pallas import tpu_sc as plsc

B, C, D = 16, 512, 4096
LANES = 16
NW = 32
ROWS = B * C
CH = 4
RPW = ROWS // NW
NCH = RPW // CH


def _gather_add(x2d, gidx3):
    mesh = plsc.VectorSubcoreMesh(core_axis_name="c", subcore_axis_name="s")

    @functools.partial(
        pl.kernel,
        out_type=jax.ShapeDtypeStruct((ROWS, D), jnp.float32),
        mesh=mesh,
        scratch_types=[
            pltpu.VMEM((NCH, CH), jnp.int32),
            [pltpu.VMEM((CH, D), jnp.float32) for _ in range(2)],
            [pltpu.VMEM((CH, D), jnp.float32) for _ in range(2)],
            [pltpu.VMEM((CH, D), jnp.float32) for _ in range(2)],
            [pltpu.SemaphoreType.DMA for _ in range(2)],
            [pltpu.SemaphoreType.DMA for _ in range(2)],
            [pltpu.SemaphoreType.DMA for _ in range(2)],
        ],
    )
    def sc_kernel(x_hbm, gidx_hbm, out_hbm, idx_v, gbufs, ibufs, obufs,
                  sem_g, sem_i, sem_o):
        wid = lax.axis_index("s") * 2 + lax.axis_index("c")
        wbase = wid * RPW
        pltpu.sync_copy(gidx_hbm.at[wid], idx_v)

        def start_in(k, slot):
            pltpu.async_copy(x_hbm.at[idx_v.at[k]], gbufs[slot], sem_g[slot])
            pltpu.async_copy(
                x_hbm.at[pl.ds(wbase + k * CH, CH)], ibufs[slot], sem_i[slot]
            )

        def wait_in(slot):
            pltpu.make_async_copy(x_hbm.at[pl.ds(0, CH)], gbufs[slot],
                                  sem_g[slot]).wait()
            pltpu.make_async_copy(x_hbm.at[pl.ds(0, CH)], ibufs[slot],
                                  sem_i[slot]).wait()

        def add_chunk(slot):
            gbuf, ibuf, obuf = gbufs[slot], ibufs[slot], obufs[slot]

            def add_body(j, c2):
                off = j * LANES
                for r in range(CH):
                    obuf[r, pl.ds(off, LANES)] = (
                        ibuf[r, pl.ds(off, LANES)] + gbuf[r, pl.ds(off, LANES)]
                    )
                return c2

            lax.fori_loop(0, D // LANES, add_body, 0)

        def start_out(k, slot):
            pltpu.async_copy(
                obufs[slot], out_hbm.at[pl.ds(wbase + k * CH, CH)], sem_o[slot]
            )

        def wait_out(k, slot):
            pltpu.make_async_copy(obufs[slot],
                                  out_hbm.at[pl.ds(wbase + k * CH, CH)],
                                  sem_o[slot]).wait()

        start_in(0, 0)
        start_in(1, 1)

        def pipe_body(i, carry):
            k0 = i * 2

            def step(k, slot):
                wait_in(slot)

                @pl.when(k >= 2)
                def _():
                    wait_out(k - 2, slot)

                add_chunk(slot)
                start_out(k, slot)

                @pl.when(k + 2 < NCH)
                def _():
                    start_in(k + 2, slot)

            step(k0, 0)
            step(k0 + 1, 1)
            return carry

        lax.fori_loop(0, NCH // 2, pipe_body, 0)
        wait_out(NCH - 2, 0)
        wait_out(NCH - 1, 1)

    return sc_kernel(x2d, gidx3)


def kernel(x, W1, b1, W2, b2):
    pooled = jnp.mean(x, axis=2)
    h = pooled @ W1 + b1
    h = jnp.where(h > 0, h, 0.01 * h)
    scores = jax.nn.sigmoid(h @ W2 + b2)
    ch_order = jnp.argsort(-scores, axis=1)

    gidx = ch_order.astype(jnp.int32) + (jnp.arange(B, dtype=jnp.int32) * C)[:, None]
    out2d = _gather_add(x.reshape(ROWS, D), gidx.reshape(NW, NCH, CH))
    return out2d.reshape(B, C, D)
